# Initial kernel scaffold; baseline (speedup 1.0000x reference)
import jax
import jax.numpy as jnp
from jax.experimental import pallas as pl


def kernel(key, extend):
    raise NotImplementedError("write your pallas kernel here")


# speedup vs baseline: 57.3522x; 57.3522x over previous
import functools

import jax
import jax.numpy as jnp
from jax import lax
from jax.experimental import pallas as pl
from jax.experimental.pallas import tpu as pltpu
from jax.experimental.pallas import tpu_sc as plsc

N = 16384
L = 16
NT = 16
OWN = 65536
BLK = N // NT
NB = BLK // L
NBLOCKS = N // L
GCHUNK = 128

_mesh = plsc.VectorSubcoreMesh(core_axis_name="c", subcore_axis_name="s")


@functools.partial(
    pl.kernel,
    out_type=jax.ShapeDtypeStruct((N,), jnp.int32),
    mesh=_mesh,
    scratch_types=[
        pltpu.VMEM((N,), jnp.int32),
        pltpu.VMEM((OWN,), jnp.int32),
        pltpu.VMEM((BLK,), jnp.int32),
        pltpu.VMEM((BLK,), jnp.int32),
        pltpu.VMEM((BLK,), jnp.int32),
        pltpu.VMEM((NT * L,), jnp.int32),
        pltpu.VMEM((L,), jnp.int32),
        pltpu.VMEM((N,), jnp.int32),
        pltpu.HBM((NT * OWN,), jnp.int32),
        pltpu.VMEM_SHARED((N,), jnp.int32),
        pltpu.VMEM_SHARED((NT * L,), jnp.int32),
        pltpu.SemaphoreType.DMA,
    ],
    compiler_params=pltpu.CompilerParams(needs_layout_passes=False),
)
def _hash_rank_sc(key_ref, out_ref, keys_v, tbl_v, fp_v, cl_v, res_v, tot_v,
                  tmp_v, cf_v, tbl_hbm, c_sh, tot_sh, sem):
    cid = lax.axis_index("c")
    sid = lax.axis_index("s").astype(jnp.int32)

    @pl.when(cid == 0)
    def _():
        iota = lax.iota(jnp.int32, L)
        base = sid * BLK

        pltpu.sync_copy(key_ref, keys_v)

        def p1(b, bb):
            v = keys_v[pl.ds(bb * L, L)]
            cnt, _ = plsc.scan_count(v)
            m = (cnt == 1) & ((v >> 16) == sid)
            pos = bb * L + iota
            plsc.store_scatter(tbl_v, [v & (OWN - 1)], pos, mask=m)
            return bb - 1
        lax.fori_loop(0, NBLOCKS, p1, jnp.int32(NBLOCKS - 1))

        pltpu.sync_copy(tbl_v, tbl_hbm.at[pl.ds(sid * OWN, OWN)])
        plsc.subcore_barrier()

        for j in range(BLK // GCHUNK):
            pltpu.async_copy(
                tbl_hbm.at[keys_v.at[pl.ds(base + j * GCHUNK, GCHUNK)]],
                fp_v.at[pl.ds(j * GCHUNK, GCHUNK)], sem).wait()

        def p3(b, carry):
            off, acc = carry
            fp = fp_v[pl.ds(off, L)]
            isf = (fp == base + off + iota).astype(jnp.int32)
            cl_v[pl.ds(off, L)] = plsc.cumsum(isf) + acc
            return off + L, acc + jnp.sum(isf, dtype=jnp.int32)
        _, total = lax.fori_loop(0, NB, p3, (jnp.int32(0), jnp.int32(0)))

        tmp_v[...] = jnp.zeros((L,), jnp.int32) + total
        pltpu.sync_copy(tmp_v, tot_sh.at[pl.ds(sid * L, L)])
        plsc.subcore_barrier()

        pltpu.sync_copy(tot_sh, tot_v)
        totals = plsc.load_gather(tot_v, [iota * L])
        my_off = jnp.sum(jnp.where(iota < sid, totals, 0), dtype=jnp.int32)

        def p4(b, off):
            cl_v[pl.ds(off, L)] = cl_v[pl.ds(off, L)] + (my_off - 1)
            return off + L
        lax.fori_loop(0, NB, p4, jnp.int32(0))
        pltpu.sync_copy(cl_v, c_sh.at[pl.ds(base, BLK)])
        plsc.subcore_barrier()

        pltpu.sync_copy(c_sh, cf_v)

        def p5(b, off):
            fp = fp_v[pl.ds(off, L)]
            res_v[pl.ds(off, L)] = plsc.load_gather(cf_v, [fp])
            return off + L
        lax.fori_loop(0, NB, p5, jnp.int32(0))
        pltpu.sync_copy(res_v, out_ref.at[pl.ds(base, BLK)])


def kernel(key, extend):
    r = _hash_rank_sc(key.astype(jnp.int32)).astype(key.dtype)
    return jnp.where(extend != 0, r, jnp.zeros_like(r))


# speedup vs baseline: 67.1574x; 1.1710x over previous
import functools

import jax
import jax.numpy as jnp
from jax import lax
from jax.experimental import pallas as pl
from jax.experimental.pallas import tpu as pltpu
from jax.experimental.pallas import tpu_sc as plsc

N = 16384
L = 16
NT = 16
OWN = 65536
BLK = N // NT
NB = BLK // L
NBLOCKS = N // L
GCHUNK = 128
P1_UNROLL = 4

_mesh = plsc.VectorSubcoreMesh(core_axis_name="c", subcore_axis_name="s")


@functools.partial(
    pl.kernel,
    out_type=jax.ShapeDtypeStruct((N,), jnp.int32),
    mesh=_mesh,
    scratch_types=[
        pltpu.VMEM((N,), jnp.int32),
        pltpu.VMEM((OWN,), jnp.int32),
        pltpu.VMEM((BLK,), jnp.int32),
        pltpu.VMEM((BLK,), jnp.int32),
        pltpu.VMEM((BLK,), jnp.int32),
        pltpu.VMEM((NT * L,), jnp.int32),
        pltpu.VMEM((L,), jnp.int32),
        pltpu.VMEM((N,), jnp.int32),
        pltpu.HBM((NT * OWN,), jnp.int32),
        pltpu.VMEM_SHARED((N,), jnp.int32),
        pltpu.VMEM_SHARED((NT * L,), jnp.int32),
        pltpu.SemaphoreType.DMA,
    ],
    compiler_params=pltpu.CompilerParams(needs_layout_passes=False),
)
def _hash_rank_sc(key_ref, out_ref, keys_v, tbl_v, fp_v, cl_v, res_v, tot_v,
                  tmp_v, cf_v, tbl_hbm, c_sh, tot_sh, sem):
    cid = lax.axis_index("c")
    sid = lax.axis_index("s").astype(jnp.int32)

    @pl.when(cid == 0)
    def _():
        iota = lax.iota(jnp.int32, L)
        base = sid * BLK

        pltpu.sync_copy(key_ref, keys_v)

        revio = lax.rev(iota, (0,))

        def p1(b, bb):
            for u in range(P1_UNROLL):
                bbu = bb - u
                vr = lax.rev(keys_v[pl.ds(bbu * L, L)], (0,))
                m = (vr >> 16) == sid
                posr = bbu * L + revio
                plsc.store_scatter(tbl_v, [vr & (OWN - 1)], posr, mask=m)
            return bb - P1_UNROLL
        lax.fori_loop(0, NBLOCKS // P1_UNROLL, p1, jnp.int32(NBLOCKS - 1))

        pltpu.sync_copy(tbl_v, tbl_hbm.at[pl.ds(sid * OWN, OWN)])
        plsc.subcore_barrier()

        for j in range(BLK // GCHUNK):
            pltpu.async_copy(
                tbl_hbm.at[keys_v.at[pl.ds(base + j * GCHUNK, GCHUNK)]],
                fp_v.at[pl.ds(j * GCHUNK, GCHUNK)], sem).wait()

        def p3(b, carry):
            off, acc = carry
            fp = fp_v[pl.ds(off, L)]
            isf = (fp == base + off + iota).astype(jnp.int32)
            cl_v[pl.ds(off, L)] = plsc.cumsum(isf) + acc
            return off + L, acc + jnp.sum(isf, dtype=jnp.int32)
        _, total = lax.fori_loop(0, NB, p3, (jnp.int32(0), jnp.int32(0)))

        tmp_v[...] = jnp.zeros((L,), jnp.int32) + total
        pltpu.sync_copy(tmp_v, tot_sh.at[pl.ds(sid * L, L)])
        plsc.subcore_barrier()

        pltpu.sync_copy(tot_sh, tot_v)
        totals = plsc.load_gather(tot_v, [iota * L])
        my_off = jnp.sum(jnp.where(iota < sid, totals, 0), dtype=jnp.int32)

        def p4(b, off):
            cl_v[pl.ds(off, L)] = cl_v[pl.ds(off, L)] + (my_off - 1)
            return off + L
        lax.fori_loop(0, NB, p4, jnp.int32(0))
        pltpu.sync_copy(cl_v, c_sh.at[pl.ds(base, BLK)])
        plsc.subcore_barrier()

        pltpu.sync_copy(c_sh, cf_v)

        def p5(b, off):
            fp = fp_v[pl.ds(off, L)]
            res_v[pl.ds(off, L)] = plsc.load_gather(cf_v, [fp])
            return off + L
        lax.fori_loop(0, NB, p5, jnp.int32(0))
        pltpu.sync_copy(res_v, out_ref.at[pl.ds(base, BLK)])


def kernel(key, extend):
    r = _hash_rank_sc(key.astype(jnp.int32)).astype(key.dtype)
    return jnp.where(extend != 0, r, jnp.zeros_like(r))


# speedup vs baseline: 74.4313x; 1.1083x over previous
import functools

import jax
import jax.numpy as jnp
from jax import lax
from jax.experimental import pallas as pl
from jax.experimental.pallas import tpu as pltpu
from jax.experimental.pallas import tpu_sc as plsc

N = 16384
L = 16
NT = 16
OWN = 65536
BLK = N // NT
NB = BLK // L
NBLOCKS = N // L
GCHUNK = 128
P1_UNROLL = 4
P5_UNROLL = 4

_mesh = plsc.VectorSubcoreMesh(core_axis_name="c", subcore_axis_name="s")


@functools.partial(
    pl.kernel,
    out_type=jax.ShapeDtypeStruct((N,), jnp.int32),
    mesh=_mesh,
    scratch_types=[
        pltpu.VMEM((N,), jnp.int32),
        pltpu.VMEM((OWN,), jnp.int32),
        pltpu.VMEM((BLK,), jnp.int32),
        pltpu.VMEM((BLK,), jnp.int32),
        pltpu.VMEM((BLK,), jnp.int32),
        pltpu.VMEM((NT * L,), jnp.int32),
        pltpu.VMEM((L,), jnp.int32),
        pltpu.VMEM((N,), jnp.int32),
        pltpu.HBM((NT * OWN,), jnp.int32),
        pltpu.VMEM_SHARED((N,), jnp.int32),
        pltpu.VMEM_SHARED((NT * L,), jnp.int32),
        pltpu.SemaphoreType.DMA,
    ],
    compiler_params=pltpu.CompilerParams(needs_layout_passes=False),
)
def _hash_rank_sc(key_ref, out_ref, keys_v, tbl_v, fp_v, cl_v, res_v, tot_v,
                  tmp_v, cf_v, tbl_hbm, c_sh, tot_sh, sem):
    cid = lax.axis_index("c")
    sid = lax.axis_index("s").astype(jnp.int32)

    @pl.when(cid == 0)
    def _():
        iota = lax.iota(jnp.int32, L)
        base = sid * BLK

        pltpu.sync_copy(key_ref, keys_v)

        revio = lax.rev(iota, (0,))

        def p1(b, bb):
            for u in range(P1_UNROLL):
                bbu = bb - u
                vr = lax.rev(keys_v[pl.ds(bbu * L, L)], (0,))
                m = (vr >> 16) == sid
                posr = bbu * L + revio
                plsc.store_scatter(tbl_v, [vr & (OWN - 1)], posr, mask=m)
            return bb - P1_UNROLL
        lax.fori_loop(0, NBLOCKS // P1_UNROLL, p1, jnp.int32(NBLOCKS - 1))

        pltpu.sync_copy(tbl_v, tbl_hbm.at[pl.ds(sid * OWN, OWN)])
        plsc.subcore_barrier()

        descs = [
            pltpu.async_copy(
                tbl_hbm.at[keys_v.at[pl.ds(base + j * GCHUNK, GCHUNK)]],
                fp_v.at[pl.ds(j * GCHUNK, GCHUNK)], sem)
            for j in range(BLK // GCHUNK)
        ]
        for d in descs:
            d.wait()

        def p3(b, carry):
            off, acc = carry
            fp = fp_v[pl.ds(off, L)]
            isf = (fp == base + off + iota).astype(jnp.int32)
            cl_v[pl.ds(off, L)] = plsc.cumsum(isf) + acc
            return off + L, acc + jnp.sum(isf, dtype=jnp.int32)
        _, total = lax.fori_loop(0, NB, p3, (jnp.int32(0), jnp.int32(0)))

        tmp_v[...] = jnp.zeros((L,), jnp.int32) + total
        pltpu.sync_copy(tmp_v, tot_sh.at[pl.ds(sid * L, L)])
        plsc.subcore_barrier()

        pltpu.sync_copy(tot_sh, tot_v)
        totals = plsc.load_gather(tot_v, [iota * L])
        my_off = jnp.sum(jnp.where(iota < sid, totals, 0), dtype=jnp.int32)

        def p4(b, off):
            cl_v[pl.ds(off, L)] = cl_v[pl.ds(off, L)] + (my_off - 1)
            return off + L
        lax.fori_loop(0, NB, p4, jnp.int32(0))
        pltpu.sync_copy(cl_v, c_sh.at[pl.ds(base, BLK)])
        plsc.subcore_barrier()

        pltpu.sync_copy(c_sh, cf_v)

        def p5(b, off):
            for u in range(P5_UNROLL):
                o = off + u * L
                fp = fp_v[pl.ds(o, L)]
                res_v[pl.ds(o, L)] = plsc.load_gather(cf_v, [fp])
            return off + P5_UNROLL * L
        lax.fori_loop(0, NB // P5_UNROLL, p5, jnp.int32(0))
        pltpu.sync_copy(res_v, out_ref.at[pl.ds(base, BLK)])


def kernel(key, extend):
    r = _hash_rank_sc(key.astype(jnp.int32)).astype(key.dtype)
    return jnp.where(extend != 0, r, jnp.zeros_like(r))
